# Initial kernel scaffold; baseline (speedup 1.0000x reference)
import jax
import jax.numpy as jnp
from jax.experimental import pallas as pl


def kernel(gt_depth, pred_depth, gt_index, pred_index):
    raise NotImplementedError("write your pallas kernel here")


# speedup vs baseline: 7.7721x; 7.7721x over previous
import functools

import jax
import jax.numpy as jnp
from jax import lax
from jax.experimental import pallas as pl
from jax.experimental.pallas import tpu as pltpu
from jax.experimental.pallas import tpu_sc as plsc

H, W = 384, 512
FX, FY = 518.8579, 519.4696
HW = H * W
NUM = 100000
NS = 15000
NS_PAD = 16384
PER_TILE = NS_PAD // 32
K_DROP = 15000
N_KEEP = 4 * NS - K_DROP
ROWS_SUB = 128
LANES = 128


def _sample_position_consts():
    skey = jax.random.key(42)
    sk1, sk2, sk3 = jax.random.split(skey, 3)
    ps = [jnp.pad(jax.random.randint(sk, (NS,), 0, NUM), (0, NS_PAD - NS))
          for sk in (sk1, sk2, sk3)]
    rows = []
    for g in range(2):
        for c in range(2):
            for k in range(3):
                rows.append(ps[k] + (g * 2 + c) * NUM)
    return jnp.stack(rows).astype(jnp.int32)


def _sc_gather(depth_flat, idx_flat, pidx):
    mesh = plsc.VectorSubcoreMesh(
        core_axis_name="c", subcore_axis_name="s", num_cores=2, num_subcores=16
    )

    @functools.partial(
        pl.kernel,
        out_type=(
            jax.ShapeDtypeStruct((24, NS_PAD), jnp.float32),
            jax.ShapeDtypeStruct((12, NS_PAD), jnp.int32),
        ),
        mesh=mesh,
        compiler_params=pltpu.CompilerParams(use_tc_tiling_on_sc=False),
        scratch_types=[
            pltpu.VMEM((12, PER_TILE), jnp.int32),
            pltpu.VMEM((12, PER_TILE), jnp.int32),
            pltpu.VMEM((24, PER_TILE), jnp.int32),
            pltpu.VMEM((24, PER_TILE), jnp.float32),
            pltpu.SemaphoreType.DMA,
        ],
    )
    def sc_kernel(depth_hbm, idx_hbm, pidx_hbm, d_out, yx_out,
                  pidx_v, yx_v, addr_v, d_v, sem):
        wid = lax.axis_index("s") * 2 + lax.axis_index("c")
        base = wid * PER_TILE
        pltpu.sync_copy(pidx_hbm.at[:, pl.ds(base, PER_TILE)], pidx_v)
        for r in range(12):
            pltpu.async_copy(idx_hbm.at[pidx_v.at[r]], yx_v.at[r], sem).wait()

        def addr_body(j, carry):
            sl = pl.ds(j * 16, 16)
            for g in range(2):
                for k in range(3):
                    y = yx_v[g * 6 + k, sl]
                    x = yx_v[g * 6 + 3 + k, sl]
                    flat = y * W + x
                    for b in range(4):
                        addr_v[(g * 4 + b) * 3 + k, sl] = flat + (g * 4 + b) * HW
            return carry

        lax.fori_loop(0, PER_TILE // 16, addr_body, 0)
        for r in range(24):
            pltpu.async_copy(depth_hbm.at[addr_v.at[r]], d_v.at[r], sem).wait()
        pltpu.sync_copy(d_v, d_out.at[:, pl.ds(base, PER_TILE)])
        pltpu.sync_copy(yx_v, yx_out.at[:, pl.ds(base, PER_TILE)])

    return sc_kernel(depth_flat, idx_flat, pidx)


def _tc_loss_body(d_ref, yx_ref, out_ref):
    normals = [[None] * 4, [None] * 4]
    for g in range(2):
        yc = [yx_ref[g * 6 + k].astype(jnp.float32) - float(H // 2)
              for k in range(3)]
        xc = [yx_ref[g * 6 + 3 + k].astype(jnp.float32) - float(W // 2)
              for k in range(3)]
        for b in range(4):
            pts = []
            for k in range(3):
                d = d_ref[(g * 4 + b) * 3 + k]
                ad = jnp.abs(d)
                pts.append((xc[k] * ad * (1.0 / FX), yc[k] * ad * (1.0 / FY), d))
            e12 = tuple(pts[1][c] - pts[0][c] for c in range(3))
            e13 = tuple(pts[2][c] - pts[0][c] for c in range(3))
            nx = e12[1] * e13[2] - e12[2] * e13[1]
            ny = e12[2] * e13[0] - e12[0] * e13[2]
            nz = e12[0] * e13[1] - e12[1] * e13[0]
            norm = jnp.sqrt(nx * nx + ny * ny + nz * nz)
            denom = norm + jnp.where(norm == 0.0, jnp.float32(0.01),
                                     jnp.float32(0.0))
            normals[g][b] = (nx / denom, ny / denom, nz / denom)

    ji = (lax.broadcasted_iota(jnp.int32, (ROWS_SUB, LANES), 0) * LANES
          + lax.broadcasted_iota(jnp.int32, (ROWS_SUB, LANES), 1))
    valid = ji < NS

    losses = []
    for b in range(4):
        gt_n, dt_n = normals[0][b], normals[1][b]
        losses.append(sum(jnp.abs(gt_n[c] - dt_n[c]) for c in range(3)))

    inf_bits = jnp.int32(0x7F800000)
    mbits = [jnp.where(valid, lax.bitcast_convert_type(v, jnp.int32), inf_bits)
             for v in losses]

    def bisect(_, lohi):
        lo, hi = lohi
        mid = lo + (hi - lo) // 2
        cnt = sum(jnp.sum((mb <= mid).astype(jnp.int32)) for mb in mbits)
        ge = cnt >= K_DROP
        return jnp.where(ge, lo, mid + 1), jnp.where(ge, mid, hi)

    _, t_bits = lax.fori_loop(0, 31, bisect, (jnp.int32(0), inf_bits))
    t = lax.bitcast_convert_type(t_bits, jnp.float32)

    cnt_lt = jnp.float32(0.0)
    sum_lt = jnp.float32(0.0)
    total = jnp.float32(0.0)
    for b in range(4):
        lt = valid & (losses[b] < t)
        cnt_lt = cnt_lt + jnp.sum(lt.astype(jnp.float32))
        sum_lt = sum_lt + jnp.sum(jnp.where(lt, losses[b], 0.0))
        total = total + jnp.sum(jnp.where(valid, losses[b], 0.0))
    dropped = sum_lt + (jnp.float32(K_DROP) - cnt_lt) * t
    out_ref[0, 0] = (total - dropped) * (1.0 / N_KEEP)


def _tc_loss(d24, yx12):
    return pl.pallas_call(
        _tc_loss_body,
        out_shape=jax.ShapeDtypeStruct((1, 1), jnp.float32),
        out_specs=pl.BlockSpec(memory_space=pltpu.SMEM),
    )(d24, yx12)


def kernel(gt_depth, pred_depth, gt_index, pred_index):
    depth_flat = jnp.concatenate(
        [gt_depth.reshape(4, HW), pred_depth.reshape(4, HW)]
    ).reshape(8 * HW)
    idx_flat = jnp.concatenate(
        [gt_index.astype(jnp.int32).reshape(2 * NUM),
         pred_index.astype(jnp.int32).reshape(2 * NUM)]
    )
    d24, yx12 = _sc_gather(depth_flat, idx_flat, _sample_position_consts())
    out = _tc_loss(d24.reshape(24, ROWS_SUB, LANES),
                   yx12.reshape(12, ROWS_SUB, LANES))
    return out.reshape(())


# speedup vs baseline: 9.1032x; 1.1713x over previous
import functools

import jax
import jax.numpy as jnp
from jax import lax
from jax.experimental import pallas as pl
from jax.experimental.pallas import tpu as pltpu
from jax.experimental.pallas import tpu_sc as plsc

H, W = 384, 512
FX, FY = 518.8579, 519.4696
HW = H * W
NUM = 100000
NS = 15000
NS_PAD = 16384
PER_TILE = NS_PAD // 32
K_DROP = 15000
N_KEEP = 4 * NS - K_DROP
NTILES = 32


def _sample_position_consts():
    skey = jax.random.key(42)
    sk1, sk2, sk3 = jax.random.split(skey, 3)
    ps = [jnp.pad(jax.random.randint(sk, (NS,), 0, NUM), (0, NS_PAD - NS))
          for sk in (sk1, sk2, sk3)]
    rows = []
    for g in range(2):
        for c in range(2):
            for k in range(3):
                rows.append(ps[k] + (g * 2 + c) * NUM)
    pidx = jnp.stack(rows).astype(jnp.int32)
    return (pidx.reshape(12, NTILES, PER_TILE)
            .transpose(1, 0, 2).reshape(NTILES, 12 * PER_TILE))


def _sc_gather(depth_flat, idx_flat, pidx):
    mesh = plsc.VectorSubcoreMesh(
        core_axis_name="c", subcore_axis_name="s", num_cores=2, num_subcores=16
    )

    @functools.partial(
        pl.kernel,
        out_type=(
            jax.ShapeDtypeStruct((32, 24 * PER_TILE), jnp.float32),
            jax.ShapeDtypeStruct((32, 12 * PER_TILE), jnp.int32),
        ),
        mesh=mesh,
        compiler_params=pltpu.CompilerParams(use_tc_tiling_on_sc=False),
        scratch_types=[
            pltpu.VMEM((12 * PER_TILE,), jnp.int32),
            pltpu.VMEM((12 * PER_TILE,), jnp.int32),
            pltpu.VMEM((24 * PER_TILE,), jnp.int32),
            pltpu.VMEM((24 * PER_TILE,), jnp.float32),
            pltpu.SemaphoreType.DMA,
        ],
    )
    def sc_kernel(depth_hbm, idx_hbm, pidx_hbm, d_out, yx_out,
                  pidx_v, yx_v, addr_v, d_v, sem):
        wid = lax.axis_index("s") * 2 + lax.axis_index("c")
        pltpu.sync_copy(pidx_hbm.at[wid], pidx_v)
        pltpu.async_copy(idx_hbm.at[pidx_v], yx_v, sem).wait()

        def addr_body(j, carry):
            for g in range(2):
                for k in range(3):
                    y = yx_v[pl.ds((g * 6 + k) * PER_TILE + j * 16, 16)]
                    x = yx_v[pl.ds((g * 6 + 3 + k) * PER_TILE + j * 16, 16)]
                    flat = y * W + x
                    for b in range(4):
                        addr_v[pl.ds(((g * 4 + b) * 3 + k) * PER_TILE
                                     + j * 16, 16)] = flat + (g * 4 + b) * HW
            return carry

        lax.fori_loop(0, PER_TILE // 16, addr_body, 0)
        pltpu.async_copy(depth_hbm.at[addr_v], d_v, sem).wait()
        pltpu.sync_copy(d_v, d_out.at[wid])
        pltpu.sync_copy(yx_v, yx_out.at[wid])

    return sc_kernel(depth_flat, idx_flat, pidx)


def _tc_loss_body(d_ref, yx_ref, out_ref):
    normals = [[None] * 4, [None] * 4]
    for g in range(2):
        yc = [yx_ref[:, g * 6 + k, :].astype(jnp.float32) - float(H // 2)
              for k in range(3)]
        xc = [yx_ref[:, g * 6 + 3 + k, :].astype(jnp.float32) - float(W // 2)
              for k in range(3)]
        for b in range(4):
            pts = []
            for k in range(3):
                d = d_ref[:, (g * 4 + b) * 3 + k, :]
                ad = jnp.abs(d)
                pts.append((xc[k] * ad * (1.0 / FX), yc[k] * ad * (1.0 / FY), d))
            e12 = tuple(pts[1][c] - pts[0][c] for c in range(3))
            e13 = tuple(pts[2][c] - pts[0][c] for c in range(3))
            nx = e12[1] * e13[2] - e12[2] * e13[1]
            ny = e12[2] * e13[0] - e12[0] * e13[2]
            nz = e12[0] * e13[1] - e12[1] * e13[0]
            norm = jnp.sqrt(nx * nx + ny * ny + nz * nz)
            denom = norm + jnp.where(norm == 0.0, jnp.float32(0.01),
                                     jnp.float32(0.0))
            normals[g][b] = (nx / denom, ny / denom, nz / denom)

    ji = (lax.broadcasted_iota(jnp.int32, (NTILES, PER_TILE), 0) * PER_TILE
          + lax.broadcasted_iota(jnp.int32, (NTILES, PER_TILE), 1))
    valid = ji < NS

    losses = []
    for b in range(4):
        gt_n, dt_n = normals[0][b], normals[1][b]
        losses.append(sum(jnp.abs(gt_n[c] - dt_n[c]) for c in range(3)))

    inf_bits = jnp.int32(0x7F800000)
    mbits = [jnp.where(valid, lax.bitcast_convert_type(v, jnp.int32), inf_bits)
             for v in losses]

    def bisect(_, lohi):
        lo, hi = lohi
        mid = lo + (hi - lo) // 2
        cnt = sum(jnp.sum((mb <= mid).astype(jnp.int32)) for mb in mbits)
        ge = cnt >= K_DROP
        return jnp.where(ge, lo, mid + 1), jnp.where(ge, mid, hi)

    _, t_bits = lax.fori_loop(0, 31, bisect, (jnp.int32(0), inf_bits))
    t = lax.bitcast_convert_type(t_bits, jnp.float32)

    cnt_lt = jnp.float32(0.0)
    sum_lt = jnp.float32(0.0)
    total = jnp.float32(0.0)
    for b in range(4):
        lt = valid & (losses[b] < t)
        cnt_lt = cnt_lt + jnp.sum(lt.astype(jnp.float32))
        sum_lt = sum_lt + jnp.sum(jnp.where(lt, losses[b], 0.0))
        total = total + jnp.sum(jnp.where(valid, losses[b], 0.0))
    dropped = sum_lt + (jnp.float32(K_DROP) - cnt_lt) * t
    out_ref[0, 0] = (total - dropped) * (1.0 / N_KEEP)


def _tc_loss(d24, yx12):
    return pl.pallas_call(
        _tc_loss_body,
        out_shape=jax.ShapeDtypeStruct((1, 1), jnp.float32),
        out_specs=pl.BlockSpec(memory_space=pltpu.SMEM),
    )(d24, yx12)


def kernel(gt_depth, pred_depth, gt_index, pred_index):
    depth_flat = jnp.concatenate(
        [gt_depth.reshape(4, HW), pred_depth.reshape(4, HW)]
    ).reshape(8 * HW)
    idx_flat = jnp.concatenate(
        [gt_index.astype(jnp.int32).reshape(2 * NUM),
         pred_index.astype(jnp.int32).reshape(2 * NUM)]
    )
    d24, yx12 = _sc_gather(depth_flat, idx_flat, _sample_position_consts())
    out = _tc_loss(d24.reshape(NTILES, 24, PER_TILE),
                   yx12.reshape(NTILES, 12, PER_TILE))
    return out.reshape(())


# speedup vs baseline: 9.1645x; 1.0067x over previous
import functools

import jax
import jax.numpy as jnp
from jax import lax
from jax.experimental import pallas as pl
from jax.experimental.pallas import tpu as pltpu
from jax.experimental.pallas import tpu_sc as plsc

H, W = 384, 512
FX, FY = 518.8579, 519.4696
HW = H * W
NUM = 100000
NS = 15000
NS_PAD = 16384
PER_TILE = NS_PAD // 32
K_DROP = 15000
N_KEEP = 4 * NS - K_DROP
NTILES = 32


def _sample_position_consts():
    skey = jax.random.key(42)
    sk1, sk2, sk3 = jax.random.split(skey, 3)
    ps = [jnp.pad(jax.random.randint(sk, (NS,), 0, NUM), (0, NS_PAD - NS))
          for sk in (sk1, sk2, sk3)]
    rows = []
    for g in range(2):
        for c in range(2):
            for k in range(3):
                rows.append(ps[k] + (g * 2 + c) * NUM)
    pidx = jnp.stack(rows).astype(jnp.int32)
    return (pidx.reshape(12, NTILES, PER_TILE)
            .transpose(1, 0, 2).reshape(NTILES, 12 * PER_TILE))


def _sc_gather(depth_flat, idx_flat, pidx):
    mesh = plsc.VectorSubcoreMesh(
        core_axis_name="c", subcore_axis_name="s", num_cores=2, num_subcores=16
    )

    @functools.partial(
        pl.kernel,
        out_type=(
            jax.ShapeDtypeStruct((32, 24 * PER_TILE), jnp.float32),
            jax.ShapeDtypeStruct((32, 12 * PER_TILE), jnp.int32),
        ),
        mesh=mesh,
        compiler_params=pltpu.CompilerParams(use_tc_tiling_on_sc=False),
        scratch_types=[
            pltpu.VMEM((12 * PER_TILE,), jnp.int32),
            pltpu.VMEM((12 * PER_TILE,), jnp.int32),
            pltpu.VMEM((24 * PER_TILE,), jnp.int32),
            pltpu.VMEM((24 * PER_TILE,), jnp.float32),
            pltpu.SemaphoreType.DMA,
            pltpu.SemaphoreType.DMA,
            pltpu.SemaphoreType.DMA,
            pltpu.SemaphoreType.DMA,
        ],
    )
    def sc_kernel(depth_hbm, idx_hbm, pidx_hbm, d_out, yx_out,
                  pidx_v, yx_v, addr_v, d_v, sem_a0, sem_a1, sem_b, sem_y):
        wid = lax.axis_index("s") * 2 + lax.axis_index("c")
        half = 6 * PER_TILE
        pltpu.sync_copy(pidx_hbm.at[wid], pidx_v)
        cp_a = [
            pltpu.async_copy(idx_hbm.at[pidx_v.at[pl.ds(g * half, half)]],
                             yx_v.at[pl.ds(g * half, half)],
                             sem_a0 if g == 0 else sem_a1)
            for g in range(2)
        ]

        def addr_body(g, j):
            for k in range(3):
                y = yx_v[pl.ds((g * 6 + k) * PER_TILE + j * 16, 16)]
                x = yx_v[pl.ds((g * 6 + 3 + k) * PER_TILE + j * 16, 16)]
                flat = y * W + x
                for b in range(4):
                    addr_v[pl.ds(((g * 4 + b) * 3 + k) * PER_TILE
                                 + j * 16, 16)] = flat + (g * 4 + b) * HW

        cp_b = []
        for g in range(2):
            cp_a[g].wait()
            lax.fori_loop(0, PER_TILE // 16,
                          lambda j, c, g=g: (addr_body(g, j), c)[1], 0)
            cp_b.append(pltpu.async_copy(
                depth_hbm.at[addr_v.at[pl.ds(g * 2 * half, 2 * half)]],
                d_v.at[pl.ds(g * 2 * half, 2 * half)], sem_b))
        cp_y = pltpu.async_copy(yx_v, yx_out.at[wid], sem_y)
        for cp in cp_b:
            cp.wait()
        pltpu.sync_copy(d_v, d_out.at[wid])
        cp_y.wait()

    return sc_kernel(depth_flat, idx_flat, pidx)


def _tc_loss_body(d_ref, yx_ref, out_ref):
    normals = [[None] * 4, [None] * 4]
    for g in range(2):
        yc = [yx_ref[:, g * 6 + k, :].astype(jnp.float32) - float(H // 2)
              for k in range(3)]
        xc = [yx_ref[:, g * 6 + 3 + k, :].astype(jnp.float32) - float(W // 2)
              for k in range(3)]
        for b in range(4):
            pts = []
            for k in range(3):
                d = d_ref[:, (g * 4 + b) * 3 + k, :]
                ad = jnp.abs(d)
                pts.append((xc[k] * ad * (1.0 / FX), yc[k] * ad * (1.0 / FY), d))
            e12 = tuple(pts[1][c] - pts[0][c] for c in range(3))
            e13 = tuple(pts[2][c] - pts[0][c] for c in range(3))
            nx = e12[1] * e13[2] - e12[2] * e13[1]
            ny = e12[2] * e13[0] - e12[0] * e13[2]
            nz = e12[0] * e13[1] - e12[1] * e13[0]
            norm = jnp.sqrt(nx * nx + ny * ny + nz * nz)
            denom = norm + jnp.where(norm == 0.0, jnp.float32(0.01),
                                     jnp.float32(0.0))
            normals[g][b] = (nx / denom, ny / denom, nz / denom)

    ji = (lax.broadcasted_iota(jnp.int32, (NTILES, PER_TILE), 0) * PER_TILE
          + lax.broadcasted_iota(jnp.int32, (NTILES, PER_TILE), 1))
    valid = ji < NS

    losses = []
    for b in range(4):
        gt_n, dt_n = normals[0][b], normals[1][b]
        losses.append(sum(jnp.abs(gt_n[c] - dt_n[c]) for c in range(3)))

    inf_bits = jnp.int32(0x7F800000)
    mbits = [jnp.where(valid, lax.bitcast_convert_type(v, jnp.int32), inf_bits)
             for v in losses]

    def bisect(_, lohi):
        lo, hi = lohi
        mid = lo + (hi - lo) // 2
        cnt = sum(jnp.sum((mb <= mid).astype(jnp.int32)) for mb in mbits)
        ge = cnt >= K_DROP
        return jnp.where(ge, lo, mid + 1), jnp.where(ge, mid, hi)

    _, t_bits = lax.fori_loop(0, 31, bisect, (jnp.int32(0), inf_bits))
    t = lax.bitcast_convert_type(t_bits, jnp.float32)

    cnt_lt = jnp.float32(0.0)
    sum_lt = jnp.float32(0.0)
    total = jnp.float32(0.0)
    for b in range(4):
        lt = valid & (losses[b] < t)
        cnt_lt = cnt_lt + jnp.sum(lt.astype(jnp.float32))
        sum_lt = sum_lt + jnp.sum(jnp.where(lt, losses[b], 0.0))
        total = total + jnp.sum(jnp.where(valid, losses[b], 0.0))
    dropped = sum_lt + (jnp.float32(K_DROP) - cnt_lt) * t
    out_ref[0, 0] = (total - dropped) * (1.0 / N_KEEP)


def _tc_loss(d24, yx12):
    return pl.pallas_call(
        _tc_loss_body,
        out_shape=jax.ShapeDtypeStruct((1, 1), jnp.float32),
        out_specs=pl.BlockSpec(memory_space=pltpu.SMEM),
    )(d24, yx12)


def kernel(gt_depth, pred_depth, gt_index, pred_index):
    depth_flat = jnp.concatenate(
        [gt_depth.reshape(4, HW), pred_depth.reshape(4, HW)]
    ).reshape(8 * HW)
    idx_flat = jnp.concatenate(
        [gt_index.astype(jnp.int32).reshape(2 * NUM),
         pred_index.astype(jnp.int32).reshape(2 * NUM)]
    )
    d24, yx12 = _sc_gather(depth_flat, idx_flat, _sample_position_consts())
    out = _tc_loss(d24.reshape(NTILES, 24, PER_TILE),
                   yx12.reshape(NTILES, 12, PER_TILE))
    return out.reshape(())
